# baseline (speedup 1.0000x reference)
import functools

import jax
import jax.numpy as jnp
from jax import lax
from jax.experimental import pallas as pl
from jax.experimental.pallas import tpu as pltpu
from jax.experimental.pallas import tpu_sc as plsc

NUM_BITS = 6
N = 64


def _pack_body(bits_ref, w_ref, idx_ref):
    acc = jnp.dot(bits_ref[...], w_ref[...], preferred_element_type=jnp.float32)
    idx_ref[...] = acc.astype(jnp.int32)


def _pack_indices(bits, w, block_rows):
    batch, feat = bits.shape
    num_ccsk = feat // NUM_BITS
    grid = (batch // block_rows,)
    return pl.pallas_call(
        _pack_body,
        grid=grid,
        in_specs=[
            pl.BlockSpec((block_rows, feat), lambda i: (i, 0)),
            pl.BlockSpec((feat, num_ccsk), lambda i: (0, 0)),
        ],
        out_specs=pl.BlockSpec((block_rows, num_ccsk), lambda i: (i, 0)),
        out_shape=jax.ShapeDtypeStruct((batch, num_ccsk), jnp.int32),
    )(bits, w)


def _sc_info():
    try:
        info = plsc.get_sparse_core_info()
        return info.num_cores, info.num_subcores
    except Exception:
        return 2, 16


def _gather_rows(table, idx_flat, total_rows):
    nc, ns = _sc_info()
    nw = nc * ns
    b_per_w = total_rows // nw
    chunk = min(512, b_per_w)
    sub = min(128, chunk)
    n_sub = chunk // sub
    n_chunks = b_per_w // chunk
    mesh = plsc.VectorSubcoreMesh(core_axis_name="c", subcore_axis_name="s")

    @functools.partial(
        pl.kernel,
        out_type=jax.ShapeDtypeStruct((total_rows, N), jnp.float32),
        mesh=mesh,
        scratch_types=[
            pltpu.VMEM((b_per_w,), jnp.int32),
            pltpu.VMEM((chunk, N), jnp.float32),
            pltpu.VMEM((chunk, N), jnp.float32),
            pltpu.SemaphoreType.DMA,
            pltpu.SemaphoreType.DMA,
            pltpu.SemaphoreType.DMA,
        ],
        compiler_params=pltpu.CompilerParams(use_tc_tiling_on_sc=False),
    )
    def gather_kernel(table_hbm, idx_hbm, out_hbm, idx_v, rows0, rows1, g0, g1, ssem):
        wid = lax.axis_index("s") * nc + lax.axis_index("c")
        base = wid * b_per_w
        pltpu.sync_copy(idx_hbm.at[pl.ds(base, b_per_w)], idx_v)

        def fire_gather(c, rows_v, sem):
            for k in range(n_sub):
                pltpu.async_copy(
                    table_hbm.at[idx_v.at[pl.ds(c * chunk + k * sub, sub)]],
                    rows_v.at[pl.ds(k * sub, sub)],
                    sem,
                )

        def wait_gather(rows_v, sem):
            pltpu.make_async_copy(
                out_hbm.at[pl.ds(base, chunk)], rows_v, sem
            ).wait()

        def emit_chunk(c, rows_v, gsem):
            wait_gather(rows_v, gsem)
            out_cp = pltpu.async_copy(
                rows_v, out_hbm.at[pl.ds(base + c * chunk, chunk)], ssem
            )
            out_cp.wait()

            @pl.when(c + 2 < n_chunks)
            def _():
                fire_gather(c + 2, rows_v, gsem)

        fire_gather(0, rows0, g0)
        if n_chunks > 1:
            fire_gather(1, rows1, g1)

        def body(t, carry):
            a = 2 * t
            emit_chunk(a, rows0, g0)
            emit_chunk(a + 1, rows1, g1)
            return carry

        lax.fori_loop(0, n_chunks // 2, body, 0)

    return gather_kernel(table, idx_flat)


def kernel(inputs, mapping_array):
    batch, feat = inputs.shape
    num_ccsk = feat // NUM_BITS

    shifts = (2 ** jnp.arange(NUM_BITS - 1, -1, -1, dtype=jnp.float32))
    w = jnp.zeros((feat, num_ccsk), jnp.float32)
    cols = jnp.repeat(jnp.arange(num_ccsk), NUM_BITS)
    rows = jnp.arange(feat)
    w = w.at[rows, cols].set(jnp.tile(shifts, num_ccsk))

    idx = _pack_indices(inputs, w, block_rows=512)
    idx_flat = idx.reshape(batch * num_ccsk)
    out = _gather_rows(mapping_array, idx_flat, batch * num_ccsk)
    return out.reshape(batch, num_ccsk * N)


# speedup vs baseline: 2.2467x; 2.2467x over previous
import functools

import jax
import jax.numpy as jnp
from jax import lax
from jax.experimental import pallas as pl
from jax.experimental.pallas import tpu as pltpu
from jax.experimental.pallas import tpu_sc as plsc

NUM_BITS = 6
N = 64


def _pair_table_body(map_ref, out_ref):
    block = out_ref.shape[0]
    base = pl.program_id(0) * block
    v = base + lax.broadcasted_iota(jnp.int32, (block, 1), 0)
    hi = v >> NUM_BITS
    lo = v & (N - 1)
    cols = lax.broadcasted_iota(jnp.int32, (1, N), 1)
    oh_hi = (hi == cols).astype(jnp.float32)
    oh_lo = (lo == cols).astype(jnp.float32)
    m = map_ref[...]
    out_ref[:, :N] = jnp.dot(oh_hi, m, preferred_element_type=jnp.float32)
    out_ref[:, N:] = jnp.dot(oh_lo, m, preferred_element_type=jnp.float32)


def _pair_table(mapping, block=512):
    v_total = N * N
    grid = (v_total // block,)
    return pl.pallas_call(
        _pair_table_body,
        grid=grid,
        in_specs=[pl.BlockSpec((N, N), lambda i: (0, 0))],
        out_specs=pl.BlockSpec((block, 2 * N), lambda i: (i, 0)),
        out_shape=jax.ShapeDtypeStruct((v_total, 2 * N), jnp.float32),
    )(mapping)


def _pack_body(bits_ref, w_ref, idx_ref):
    acc = jnp.dot(bits_ref[...], w_ref[...], preferred_element_type=jnp.float32)
    idx_ref[...] = acc.astype(jnp.int32)


def _pack_indices(bits, w, block_rows):
    batch, feat = bits.shape
    num_pair = feat // (2 * NUM_BITS)
    grid = (batch // block_rows,)
    return pl.pallas_call(
        _pack_body,
        grid=grid,
        in_specs=[
            pl.BlockSpec((block_rows, feat), lambda i: (i, 0)),
            pl.BlockSpec((feat, num_pair), lambda i: (0, 0)),
        ],
        out_specs=pl.BlockSpec((block_rows, num_pair), lambda i: (i, 0)),
        out_shape=jax.ShapeDtypeStruct((batch, num_pair), jnp.int32),
    )(bits, w)


def _sc_info():
    try:
        info = plsc.get_sparse_core_info()
        return info.num_cores, info.num_subcores
    except Exception:
        return 2, 16


def _gather_rows(table2, idx_flat, total_rows, row_w):
    nc, ns = _sc_info()
    nw = nc * ns
    b_per_w = total_rows // nw
    chunk = min(256, b_per_w)
    sub = min(128, chunk)
    n_sub = chunk // sub
    n_chunks = b_per_w // chunk
    mesh = plsc.VectorSubcoreMesh(core_axis_name="c", subcore_axis_name="s")

    @functools.partial(
        pl.kernel,
        out_type=jax.ShapeDtypeStruct((total_rows, row_w), jnp.float32),
        mesh=mesh,
        scratch_types=[
            pltpu.VMEM((b_per_w,), jnp.int32),
            pltpu.VMEM((chunk, row_w), jnp.float32),
            pltpu.VMEM((chunk, row_w), jnp.float32),
            pltpu.SemaphoreType.DMA,
            pltpu.SemaphoreType.DMA,
            pltpu.SemaphoreType.DMA,
        ],
        compiler_params=pltpu.CompilerParams(use_tc_tiling_on_sc=False),
    )
    def gather_kernel(table_hbm, idx_hbm, out_hbm, idx_v, rows0, rows1, g0, g1, ssem):
        wid = lax.axis_index("s") * nc + lax.axis_index("c")
        base = wid * b_per_w
        pltpu.sync_copy(idx_hbm.at[pl.ds(base, b_per_w)], idx_v)

        def fire_gather(c, rows_v, sem):
            for k in range(n_sub):
                pltpu.async_copy(
                    table_hbm.at[idx_v.at[pl.ds(c * chunk + k * sub, sub)]],
                    rows_v.at[pl.ds(k * sub, sub)],
                    sem,
                )

        def wait_gather(rows_v, sem):
            pltpu.make_async_copy(
                out_hbm.at[pl.ds(base, chunk)], rows_v, sem
            ).wait()

        def emit_chunk(c, rows_v, gsem):
            wait_gather(rows_v, gsem)
            out_cp = pltpu.async_copy(
                rows_v, out_hbm.at[pl.ds(base + c * chunk, chunk)], ssem
            )
            out_cp.wait()

            @pl.when(c + 2 < n_chunks)
            def _():
                fire_gather(c + 2, rows_v, gsem)

        fire_gather(0, rows0, g0)
        if n_chunks > 1:
            fire_gather(1, rows1, g1)

        def body(t, carry):
            a = 2 * t
            emit_chunk(a, rows0, g0)
            emit_chunk(a + 1, rows1, g1)
            return carry

        lax.fori_loop(0, n_chunks // 2, body, 0)

    return gather_kernel(table2, idx_flat)


def kernel(inputs, mapping_array):
    batch, feat = inputs.shape
    pair_bits = 2 * NUM_BITS
    num_pair = feat // pair_bits

    shifts = 2 ** jnp.arange(pair_bits - 1, -1, -1, dtype=jnp.float32)
    w = jnp.zeros((feat, num_pair), jnp.float32)
    cols = jnp.repeat(jnp.arange(num_pair), pair_bits)
    rows = jnp.arange(feat)
    w = w.at[rows, cols].set(jnp.tile(shifts, num_pair))

    table2 = _pair_table(mapping_array)
    idx2 = _pack_indices(inputs, w, block_rows=512)
    idx_flat = idx2.reshape(batch * num_pair)
    out = _gather_rows(table2, idx_flat, batch * num_pair, 2 * N)
    return out.reshape(batch, num_pair * 2 * N)


# speedup vs baseline: 2.6599x; 1.1839x over previous
import functools

import jax
import jax.numpy as jnp
from jax import lax
from jax.experimental import pallas as pl
from jax.experimental.pallas import tpu as pltpu
from jax.experimental.pallas import tpu_sc as plsc

NUM_BITS = 6
N = 64


def _pair_table_body(map_ref, out_ref):
    block = out_ref.shape[0]
    base = pl.program_id(0) * block
    v = base + lax.broadcasted_iota(jnp.int32, (block, 1), 0)
    hi = v >> NUM_BITS
    lo = v & (N - 1)
    cols = lax.broadcasted_iota(jnp.int32, (1, N), 1)
    oh_hi = (hi == cols).astype(jnp.float32)
    oh_lo = (lo == cols).astype(jnp.float32)
    m = map_ref[...]
    out_ref[:, :N] = jnp.dot(oh_hi, m, preferred_element_type=jnp.float32)
    out_ref[:, N:] = jnp.dot(oh_lo, m, preferred_element_type=jnp.float32)


def _pair_table(mapping, block=512):
    v_total = N * N
    grid = (v_total // block,)
    return pl.pallas_call(
        _pair_table_body,
        grid=grid,
        in_specs=[pl.BlockSpec((N, N), lambda i: (0, 0))],
        out_specs=pl.BlockSpec((block, 2 * N), lambda i: (i, 0)),
        out_shape=jax.ShapeDtypeStruct((v_total, 2 * N), jnp.float32),
    )(mapping)


def _pack_body(bits_ref, w_ref, idx_ref):
    acc = jnp.dot(bits_ref[...], w_ref[...], preferred_element_type=jnp.float32)
    idx_ref[...] = acc.astype(jnp.int32)


def _pack_indices(bits, w, block_rows):
    batch, feat = bits.shape
    num_pair = feat // (2 * NUM_BITS)
    grid = (batch // block_rows,)
    return pl.pallas_call(
        _pack_body,
        grid=grid,
        in_specs=[
            pl.BlockSpec((block_rows, feat), lambda i: (i, 0)),
            pl.BlockSpec((feat, num_pair), lambda i: (0, 0)),
        ],
        out_specs=pl.BlockSpec((block_rows, num_pair), lambda i: (i, 0)),
        out_shape=jax.ShapeDtypeStruct((batch, num_pair), jnp.int32),
    )(bits, w)


def _sc_info():
    try:
        info = plsc.get_sparse_core_info()
        return info.num_cores, info.num_subcores
    except Exception:
        return 2, 16


def _gather_rows(table2, idx_flat, total_rows, row_w):
    nc, ns = _sc_info()
    nw = nc * ns
    b_per_w = total_rows // nw
    chunk = min(256, b_per_w)
    sub = min(128, chunk)
    n_sub = chunk // sub
    n_chunks = b_per_w // chunk
    mesh = plsc.VectorSubcoreMesh(core_axis_name="c", subcore_axis_name="s")

    @functools.partial(
        pl.kernel,
        out_type=jax.ShapeDtypeStruct((total_rows, row_w), jnp.float32),
        mesh=mesh,
        scratch_types=[
            pltpu.VMEM((b_per_w,), jnp.int32),
            pltpu.VMEM((chunk, row_w), jnp.float32),
            pltpu.VMEM((chunk, row_w), jnp.float32),
            pltpu.VMEM_SHARED((N * N, row_w), jnp.float32),
            pltpu.SemaphoreType.DMA,
            pltpu.SemaphoreType.DMA,
            pltpu.SemaphoreType.DMA,
        ],
        compiler_params=pltpu.CompilerParams(use_tc_tiling_on_sc=False),
    )
    def gather_kernel(table_hbm, idx_hbm, out_hbm, idx_v, rows0, rows1, table_sh, g0, g1, ssem):
        wid = lax.axis_index("s") * nc + lax.axis_index("c")
        base = wid * b_per_w

        @pl.when(lax.axis_index("s") == 0)
        def _():
            pltpu.sync_copy(table_hbm, table_sh)

        pltpu.sync_copy(idx_hbm.at[pl.ds(base, b_per_w)], idx_v)
        plsc.subcore_barrier()

        def fire_gather(c, rows_v, sem):
            for k in range(n_sub):
                pltpu.async_copy(
                    table_sh.at[idx_v.at[pl.ds(c * chunk + k * sub, sub)]],
                    rows_v.at[pl.ds(k * sub, sub)],
                    sem,
                )

        def wait_gather(rows_v, sem):
            pltpu.make_async_copy(
                out_hbm.at[pl.ds(base, chunk)], rows_v, sem
            ).wait()

        def emit_chunk(c, rows_v, gsem):
            wait_gather(rows_v, gsem)
            out_cp = pltpu.async_copy(
                rows_v, out_hbm.at[pl.ds(base + c * chunk, chunk)], ssem
            )
            out_cp.wait()

            @pl.when(c + 2 < n_chunks)
            def _():
                fire_gather(c + 2, rows_v, gsem)

        fire_gather(0, rows0, g0)
        if n_chunks > 1:
            fire_gather(1, rows1, g1)

        def body(t, carry):
            a = 2 * t
            emit_chunk(a, rows0, g0)
            emit_chunk(a + 1, rows1, g1)
            return carry

        lax.fori_loop(0, n_chunks // 2, body, 0)

    return gather_kernel(table2, idx_flat)


def kernel(inputs, mapping_array):
    batch, feat = inputs.shape
    pair_bits = 2 * NUM_BITS
    num_pair = feat // pair_bits

    shifts = 2 ** jnp.arange(pair_bits - 1, -1, -1, dtype=jnp.float32)
    w = jnp.zeros((feat, num_pair), jnp.float32)
    cols = jnp.repeat(jnp.arange(num_pair), pair_bits)
    rows = jnp.arange(feat)
    w = w.at[rows, cols].set(jnp.tile(shifts, num_pair))

    table2 = _pair_table(mapping_array)
    idx2 = _pack_indices(inputs, w, block_rows=512)
    idx_flat = idx2.reshape(batch * num_pair)
    out = _gather_rows(table2, idx_flat, batch * num_pair, 2 * N)
    return out.reshape(batch, num_pair * 2 * N)
